# Initial kernel scaffold; baseline (speedup 1.0000x reference)
import functools

import jax
import jax.numpy as jnp
from jax import lax
from jax.experimental import pallas as pl
from jax.experimental.pallas import tpu as pltpu
from jax.experimental.pallas import tpu_sc as plsc

_N = 10000
_E = 320000
_D = 128
_EPS = 0.3

_NC = 2
_NS = 16
_LANES = 16
_NW = _NC * _NS
_EW = _E // _NW
_B = 80
_NB = _EW // _B
_G = _B // _LANES

_STRIPE = 624
_TAIL = _N - 16 * _STRIPE
_ROWS_PER_TILE = _N // _NS

_f32 = jnp.float32
_i32 = jnp.int32

_MESH = plsc.VectorSubcoreMesh(core_axis_name="c", subcore_axis_name="s")


@functools.partial(
    pl.kernel,
    out_type=jax.ShapeDtypeStruct((_NC, _N), _f32),
    mesh=_MESH,
    scratch_types=[
        pltpu.VMEM((_B,), _i32),
        pltpu.VMEM((_B,), _f32),
        pltpu.VMEM((_STRIPE,), _f32),
        pltpu.VMEM_SHARED((_N,), _f32),
    ],
)
def _deg_kernel(ei_hbm, deg_out, dsti_v, ones_v, zb_v, deg_sh):
    cid = lax.axis_index("c")
    sid = lax.axis_index("s")
    for j in range(_B // _LANES):
        ones_v[pl.ds(j * _LANES, _LANES)] = jnp.full((_LANES,), 1.0, _f32)

    def _zfill(j, c):
        zb_v[pl.ds(j * _LANES, _LANES)] = jnp.zeros((_LANES,), _f32)
        return c

    lax.fori_loop(0, _STRIPE // _LANES, _zfill, 0)
    pltpu.sync_copy(zb_v, deg_sh.at[pl.ds(sid * _STRIPE, _STRIPE)])

    @pl.when(sid == _NS - 1)
    def _():
        pltpu.sync_copy(zb_v.at[pl.ds(0, _TAIL)],
                        deg_sh.at[pl.ds(16 * _STRIPE, _TAIL)])

    plsc.subcore_barrier()

    base = (cid * _NS + sid) * _EW

    def _body(i, c):
        pltpu.sync_copy(ei_hbm.at[1, pl.ds(base + i * _B, _B)], dsti_v)
        pltpu.sync_copy(ones_v, deg_sh.at[dsti_v], add=True)
        return c

    lax.fori_loop(0, _NB, _body, 0)
    plsc.subcore_barrier()

    pltpu.sync_copy(deg_sh.at[pl.ds(sid * _STRIPE, _STRIPE)],
                    deg_out.at[cid, pl.ds(sid * _STRIPE, _STRIPE)])

    @pl.when(sid == _NS - 1)
    def _():
        pltpu.sync_copy(deg_sh.at[pl.ds(16 * _STRIPE, _TAIL)],
                        deg_out.at[cid, pl.ds(16 * _STRIPE, _TAIL)])


@functools.partial(
    pl.kernel,
    out_type=jax.ShapeDtypeStruct((_NC, _N, _D), _f32),
    mesh=_MESH,
    scratch_types=[
        pltpu.VMEM((_N,), _f32),
        pltpu.VMEM((_N,), _f32),
        pltpu.VMEM((_N,), _f32),
        pltpu.VMEM((_B,), _i32),
        pltpu.VMEM((_B,), _i32),
        pltpu.VMEM((_B,), _f32),
        pltpu.VMEM((_B, _D), _f32),
        pltpu.VMEM((16, _D), _f32),
        pltpu.SemaphoreType.DMA,
        pltpu.VMEM_SHARED((_N, _D), _f32),
    ],
)
def _faconv_kernel(h_hbm, ei_hbm, hl_hbm, hr_hbm, dis_hbm, part_out,
                   hl_v, hr_v, dis_v, srci_v, dsti_v, w_v, rows_v, zr_v,
                   gsem, out_sh):
    cid = lax.axis_index("c")
    sid = lax.axis_index("s")

    pltpu.sync_copy(hl_hbm, hl_v)
    pltpu.sync_copy(hr_hbm, hr_v)
    pltpu.sync_copy(dis_hbm, dis_v)

    for r in range(16):
        for f in range(_D // _LANES):
            zr_v[r, pl.ds(f * _LANES, _LANES)] = jnp.zeros((_LANES,), _f32)

    rbase = sid * _ROWS_PER_TILE

    def _zero(k, c):
        pltpu.sync_copy(zr_v, out_sh.at[pl.ds(rbase + k * 16, 16)])
        return c

    lax.fori_loop(0, _ROWS_PER_TILE // 16, _zero, 0)
    pltpu.sync_copy(zr_v.at[pl.ds(0, 1)],
                    out_sh.at[pl.ds(rbase + _ROWS_PER_TILE - 1, 1)])
    plsc.subcore_barrier()

    base = (cid * _NS + sid) * _EW

    def _body(i, c):
        eb = base + i * _B
        pltpu.sync_copy(ei_hbm.at[0, pl.ds(eb, _B)], srci_v)
        pltpu.sync_copy(ei_hbm.at[1, pl.ds(eb, _B)], dsti_v)
        pltpu.async_copy(h_hbm.at[srci_v], rows_v, gsem).wait()
        for g in range(_G):
            s16 = srci_v[pl.ds(g * _LANES, _LANES)]
            d16 = dsti_v[pl.ds(g * _LANES, _LANES)]
            hld = plsc.load_gather(hl_v, [d16])
            hrs = plsc.load_gather(hr_v, [s16])
            dss = plsc.load_gather(dis_v, [s16])
            dsd = plsc.load_gather(dis_v, [d16])
            y = hld + hrs
            e = jnp.exp(jnp.abs(y) * (-2.0))
            t = (1.0 - e) / (1.0 + e)
            t = jnp.where(y < 0.0, -t, t)
            w_v[pl.ds(g * _LANES, _LANES)] = t * dss * dsd
            for l in range(_LANES):
                r = g * _LANES + l
                wl = plsc.load_gather(w_v, [jnp.full((_LANES,), r, _i32)])
                for f in range(_D // _LANES):
                    sl = pl.ds(f * _LANES, _LANES)
                    rows_v[r, sl] = rows_v[r, sl] * wl
        pltpu.sync_copy(rows_v, out_sh.at[dsti_v], add=True)
        return c

    lax.fori_loop(0, _NB, _body, 0)
    plsc.subcore_barrier()

    pltpu.sync_copy(out_sh.at[pl.ds(rbase, _ROWS_PER_TILE)],
                    part_out.at[cid, pl.ds(rbase, _ROWS_PER_TILE)])


def _tc_a_body(x_ref, w1_ref, b1_ref, al_ref, ar_ref, degp_ref,
               h_ref, hl_ref, hr_ref, dis_ref):
    xw = lax.dot_general(x_ref[...], w1_ref[...], (((1,), (1,)), ((), ())),
                         preferred_element_type=_f32)
    h = jnp.maximum(xw + b1_ref[...][None, :], 0.0)
    h_ref[...] = h
    hl_ref[...] = jnp.sum(h * al_ref[...][None, :], axis=1)
    hr_ref[...] = jnp.sum(h * ar_ref[...][None, :], axis=1)
    deg = degp_ref[0, :] + degp_ref[1, :]
    dis_ref[...] = jnp.where(deg > 0.0, 1.0 / jnp.sqrt(jnp.maximum(deg, 1.0)), 0.0)


def _tc_b_body(raw_ref, p_ref, al_ref, ar_ref, h_ref, hl_ref, hr_ref):
    h = _EPS * raw_ref[...] + p_ref[0] + p_ref[1]
    h_ref[...] = h
    hl_ref[...] = jnp.sum(h * al_ref[...][None, :], axis=1)
    hr_ref[...] = jnp.sum(h * ar_ref[...][None, :], axis=1)


def _tc_c_body(raw_ref, q_ref, w2_ref, b2_ref, o_ref):
    h = _EPS * raw_ref[...] + q_ref[0] + q_ref[1]
    z = lax.dot_general(h, w2_ref[...], (((1,), (1,)), ((), ())),
                        preferred_element_type=_f32) + b2_ref[...][None, :]
    m = jnp.max(z, axis=1, keepdims=True)
    s = jnp.log(jnp.sum(jnp.exp(z - m), axis=1, keepdims=True))
    o_ref[...] = z - m - s


_tc_a = pl.pallas_call(
    _tc_a_body,
    out_shape=[
        jax.ShapeDtypeStruct((_N, _D), _f32),
        jax.ShapeDtypeStruct((_N,), _f32),
        jax.ShapeDtypeStruct((_N,), _f32),
        jax.ShapeDtypeStruct((_N,), _f32),
    ],
)

_tc_b = pl.pallas_call(
    _tc_b_body,
    out_shape=[
        jax.ShapeDtypeStruct((_N, _D), _f32),
        jax.ShapeDtypeStruct((_N,), _f32),
        jax.ShapeDtypeStruct((_N,), _f32),
    ],
)

_tc_c = pl.pallas_call(
    _tc_c_body,
    out_shape=jax.ShapeDtypeStruct((_N, _D), _f32),
)


def kernel(x, edge_index, W1, b1, W2, b2, aL0, aR0, aL1, aR1):
    degp = _deg_kernel(edge_index)
    h, hl0, hr0, dis = _tc_a(x, W1, b1, aL0, aR0, degp)
    p0 = _faconv_kernel(h, edge_index, hl0, hr0, dis)
    h1, hl1, hr1 = _tc_b(h, p0, aL1, aR1)
    p1 = _faconv_kernel(h1, edge_index, hl1, hr1, dis)
    return _tc_c(h, p1, W2, b2)


# speedup vs baseline: 2.5417x; 2.5417x over previous
import functools

import jax
import jax.numpy as jnp
from jax import lax
from jax.experimental import pallas as pl
from jax.experimental.pallas import tpu as pltpu
from jax.experimental.pallas import tpu_sc as plsc

_N = 10000
_E = 320000
_D = 128
_EPS = 0.3

_NC = 2
_NS = 16
_LANES = 16
_NW = _NC * _NS
_EW = _E // _NW
_B = 80
_NB = _EW // _B
_G = _B // _LANES

_STRIPE = 624
_TAIL = _N - 16 * _STRIPE
_ROWS_PER_TILE = _N // _NS

_f32 = jnp.float32
_i32 = jnp.int32

_MESH = plsc.VectorSubcoreMesh(core_axis_name="c", subcore_axis_name="s",
                               num_cores=_NC, num_subcores=_NS)


_DEG_KW = dict(
    out_type=jax.ShapeDtypeStruct((_NC * _N,), _f32),
    mesh=_MESH,
    compiler_params=pltpu.CompilerParams(needs_layout_passes=False),
    scratch_types=[
        pltpu.VMEM((_B,), _i32),
        pltpu.VMEM((_B,), _f32),
        pltpu.VMEM((_STRIPE,), _f32),
        pltpu.VMEM_SHARED((_N,), _f32),
    ],
)


def _deg_body(dst_hbm, deg_out, dsti_v, ones_v, zb_v, deg_sh):
    cid = lax.axis_index("c")
    sid = lax.axis_index("s")
    for j in range(_B // _LANES):
        ones_v[pl.ds(j * _LANES, _LANES)] = jnp.full((_LANES,), 1.0, _f32)

    def _zfill(j, c):
        zb_v[pl.ds(j * _LANES, _LANES)] = jnp.zeros((_LANES,), _f32)
        return c

    lax.fori_loop(0, _STRIPE // _LANES, _zfill, 0)
    pltpu.sync_copy(zb_v, deg_sh.at[pl.ds(sid * _STRIPE, _STRIPE)])

    @pl.when(sid == _NS - 1)
    def _():
        pltpu.sync_copy(zb_v.at[pl.ds(0, _TAIL)],
                        deg_sh.at[pl.ds(16 * _STRIPE, _TAIL)])

    plsc.subcore_barrier()

    base = (cid * _NS + sid) * _EW

    def _body(i, c):
        pltpu.sync_copy(dst_hbm.at[pl.ds(base + i * _B, _B)], dsti_v)
        pltpu.sync_copy(ones_v, deg_sh.at[dsti_v], add=True)
        return c

    lax.fori_loop(0, _NB, _body, 0)
    plsc.subcore_barrier()

    pltpu.sync_copy(deg_sh.at[pl.ds(sid * _STRIPE, _STRIPE)], zb_v)
    pltpu.sync_copy(zb_v, deg_out.at[pl.ds(cid * _N + sid * _STRIPE, _STRIPE)])

    @pl.when(sid == _NS - 1)
    def _():
        pltpu.sync_copy(deg_sh.at[pl.ds(16 * _STRIPE, _TAIL)],
                        ones_v.at[pl.ds(0, _TAIL)])
        pltpu.sync_copy(ones_v.at[pl.ds(0, _TAIL)],
                        deg_out.at[pl.ds(cid * _N + 16 * _STRIPE, _TAIL)])


_deg_kernel = pl.kernel(_deg_body, **_DEG_KW)


_FACONV_KW = dict(
    out_type=jax.ShapeDtypeStruct((_NC, _N, _D), _f32),
    mesh=_MESH,
    compiler_params=pltpu.CompilerParams(needs_layout_passes=False),
    scratch_types=[
        pltpu.VMEM((_N,), _f32),
        pltpu.VMEM((_N,), _f32),
        pltpu.VMEM((_N,), _f32),
        pltpu.VMEM((_B,), _i32),
        pltpu.VMEM((_B,), _i32),
        pltpu.VMEM((_B,), _f32),
        pltpu.VMEM((_B, _D), _f32),
        pltpu.VMEM((16, _D), _f32),
        pltpu.SemaphoreType.DMA,
        pltpu.VMEM_SHARED((_N, _D), _f32),
    ],
)


def _faconv_body(h_hbm, src_hbm, dst_hbm, hl_hbm, hr_hbm, dis_hbm, part_out,
                   hl_v, hr_v, dis_v, srci_v, dsti_v, w_v, rows_v, zr_v,
                   gsem, out_sh):
    cid = lax.axis_index("c")
    sid = lax.axis_index("s")

    pltpu.sync_copy(hl_hbm, hl_v)
    pltpu.sync_copy(hr_hbm, hr_v)
    pltpu.sync_copy(dis_hbm, dis_v)

    for r in range(16):
        for f in range(_D // _LANES):
            zr_v[r, pl.ds(f * _LANES, _LANES)] = jnp.zeros((_LANES,), _f32)

    rbase = sid * _STRIPE

    def _zero(k, c):
        pltpu.sync_copy(zr_v, out_sh.at[pl.ds(rbase + k * 16, 16)])
        return c

    lax.fori_loop(0, _STRIPE // 16, _zero, 0)

    @pl.when(sid == _NS - 1)
    def _():
        pltpu.sync_copy(zr_v, out_sh.at[pl.ds(16 * _STRIPE, _TAIL)])

    plsc.subcore_barrier()

    base = (cid * _NS + sid) * _EW

    def _body(i, c):
        eb = base + i * _B
        pltpu.sync_copy(src_hbm.at[pl.ds(eb, _B)], srci_v)
        pltpu.sync_copy(dst_hbm.at[pl.ds(eb, _B)], dsti_v)
        pltpu.async_copy(h_hbm.at[srci_v], rows_v, gsem).wait()
        for g in range(_G):
            s16 = srci_v[pl.ds(g * _LANES, _LANES)]
            d16 = dsti_v[pl.ds(g * _LANES, _LANES)]
            hld = plsc.load_gather(hl_v, [d16])
            hrs = plsc.load_gather(hr_v, [s16])
            dss = plsc.load_gather(dis_v, [s16])
            dsd = plsc.load_gather(dis_v, [d16])
            y = hld + hrs
            e = jnp.exp(jnp.abs(y) * (-2.0))
            t = (1.0 - e) / (1.0 + e)
            t = jnp.where(y < 0.0, -t, t)
            w16 = t * dss * dsd
            r16 = lax.iota(_i32, _LANES) + g * _LANES
            for c in range(_D):
                cvec = jnp.full((_LANES,), c, _i32)
                col = plsc.load_gather(rows_v, [r16, cvec])
                plsc.store_scatter(rows_v, [r16, cvec], col * w16)
        pltpu.sync_copy(rows_v, out_sh.at[dsti_v], add=True)
        return c

    lax.fori_loop(0, _NB, _body, 0)
    plsc.subcore_barrier()

    def _cpout(off, cnt):
        pltpu.sync_copy(out_sh.at[pl.ds(off, cnt)], rows_v.at[pl.ds(0, cnt)])
        pltpu.sync_copy(rows_v.at[pl.ds(0, cnt)],
                        part_out.at[cid, pl.ds(off, cnt)])

    for j in range(_STRIPE // _B):
        _cpout(rbase + j * _B, _B)
    _cpout(rbase + (_STRIPE // _B) * _B, _STRIPE % _B)

    @pl.when(sid == _NS - 1)
    def _():
        _cpout(16 * _STRIPE, _TAIL)


_faconv_kernel = pl.kernel(_faconv_body, **_FACONV_KW)


def _tc_a_body(x_ref, w1_ref, b1_ref, al_ref, ar_ref, degp_ref,
               h_ref, hl_ref, hr_ref, dis_ref):
    xw = lax.dot_general(x_ref[...], w1_ref[...], (((1,), (1,)), ((), ())),
                         preferred_element_type=_f32)
    h = jnp.maximum(xw + b1_ref[...][None, :], 0.0)
    h_ref[...] = h
    hl_ref[...] = jnp.sum(h * al_ref[...][None, :], axis=1)
    hr_ref[...] = jnp.sum(h * ar_ref[...][None, :], axis=1)
    dp = degp_ref[...]
    deg = dp[:_N] + dp[_N:]
    dis_ref[...] = jnp.where(deg > 0.0, 1.0 / jnp.sqrt(jnp.maximum(deg, 1.0)), 0.0)


def _tc_b_body(raw_ref, p_ref, al_ref, ar_ref, h_ref, hl_ref, hr_ref):
    h = _EPS * raw_ref[...] + p_ref[0] + p_ref[1]
    h_ref[...] = h
    hl_ref[...] = jnp.sum(h * al_ref[...][None, :], axis=1)
    hr_ref[...] = jnp.sum(h * ar_ref[...][None, :], axis=1)


def _tc_c_body(raw_ref, q_ref, w2_ref, b2_ref, o_ref):
    h = _EPS * raw_ref[...] + q_ref[0] + q_ref[1]
    z = lax.dot_general(h, w2_ref[...], (((1,), (1,)), ((), ())),
                        preferred_element_type=_f32) + b2_ref[...][None, :]
    m = jnp.max(z, axis=1, keepdims=True)
    s = jnp.log(jnp.sum(jnp.exp(z - m), axis=1, keepdims=True))
    o_ref[...] = z - m - s


_tc_a = pl.pallas_call(
    _tc_a_body,
    out_shape=[
        jax.ShapeDtypeStruct((_N, _D), _f32),
        jax.ShapeDtypeStruct((_N,), _f32),
        jax.ShapeDtypeStruct((_N,), _f32),
        jax.ShapeDtypeStruct((_N,), _f32),
    ],
)

_tc_b = pl.pallas_call(
    _tc_b_body,
    out_shape=[
        jax.ShapeDtypeStruct((_N, _D), _f32),
        jax.ShapeDtypeStruct((_N,), _f32),
        jax.ShapeDtypeStruct((_N,), _f32),
    ],
)

_tc_c = pl.pallas_call(
    _tc_c_body,
    out_shape=jax.ShapeDtypeStruct((_N, _D), _f32),
)


def kernel(x, edge_index, W1, b1, W2, b2, aL0, aR0, aL1, aR1):
    src = edge_index[0]
    dst = edge_index[1]
    degp = _deg_kernel(dst)
    h, hl0, hr0, dis = _tc_a(x, W1, b1, aL0, aR0, degp)
    p0 = _faconv_kernel(h, src, dst, hl0, hr0, dis)
    h1, hl1, hr1 = _tc_b(h, p0, aL1, aR1)
    p1 = _faconv_kernel(h1, src, dst, hl1, hr1, dis)
    return _tc_c(h, p1, W2, b2)


# speedup vs baseline: 10.9206x; 4.2966x over previous
import functools

import numpy as np

import jax
import jax.numpy as jnp
from jax import lax
from jax.experimental import pallas as pl
from jax.experimental.pallas import tpu as pltpu
from jax.experimental.pallas import tpu_sc as plsc

_N = 10000
_E = 320000
_D = 128
_EPS = 0.3

_NC = 2
_NS = 16
_LANES = 16
_NW = _NC * _NS
_EW = _E // _NW
_B = 80
_NB = _EW // _B
_G = _B // _LANES

_STRIPE = 624
_TAIL = _N - 16 * _STRIPE
_ROWS_PER_TILE = _N // _NS

_f32 = jnp.float32
_i32 = jnp.int32

_MESH = plsc.VectorSubcoreMesh(core_axis_name="c", subcore_axis_name="s",
                               num_cores=_NC, num_subcores=_NS)


_DEG_KW = dict(
    out_type=jax.ShapeDtypeStruct((_NC * _N,), _f32),
    mesh=_MESH,
    compiler_params=pltpu.CompilerParams(needs_layout_passes=False),
    scratch_types=[
        pltpu.VMEM((_B,), _i32),
        pltpu.VMEM((_B,), _f32),
        pltpu.VMEM((_STRIPE,), _f32),
        pltpu.VMEM_SHARED((_N,), _f32),
    ],
)


def _deg_body(dst_hbm, deg_out, dsti_v, ones_v, zb_v, deg_sh):
    cid = lax.axis_index("c")
    sid = lax.axis_index("s")
    for j in range(_B // _LANES):
        ones_v[pl.ds(j * _LANES, _LANES)] = jnp.full((_LANES,), 1.0, _f32)

    def _zfill(j, c):
        zb_v[pl.ds(j * _LANES, _LANES)] = jnp.zeros((_LANES,), _f32)
        return c

    lax.fori_loop(0, _STRIPE // _LANES, _zfill, 0)
    pltpu.sync_copy(zb_v, deg_sh.at[pl.ds(sid * _STRIPE, _STRIPE)])

    @pl.when(sid == _NS - 1)
    def _():
        pltpu.sync_copy(zb_v.at[pl.ds(0, _TAIL)],
                        deg_sh.at[pl.ds(16 * _STRIPE, _TAIL)])

    plsc.subcore_barrier()

    base = (cid * _NS + sid) * _EW

    def _body(i, c):
        pltpu.sync_copy(dst_hbm.at[pl.ds(base + i * _B, _B)], dsti_v)
        pltpu.sync_copy(ones_v, deg_sh.at[dsti_v], add=True)
        return c

    lax.fori_loop(0, _NB, _body, 0)
    plsc.subcore_barrier()

    pltpu.sync_copy(deg_sh.at[pl.ds(sid * _STRIPE, _STRIPE)], zb_v)
    pltpu.sync_copy(zb_v, deg_out.at[pl.ds(cid * _N + sid * _STRIPE, _STRIPE)])

    @pl.when(sid == _NS - 1)
    def _():
        pltpu.sync_copy(deg_sh.at[pl.ds(16 * _STRIPE, _TAIL)],
                        ones_v.at[pl.ds(0, _TAIL)])
        pltpu.sync_copy(ones_v.at[pl.ds(0, _TAIL)],
                        deg_out.at[pl.ds(cid * _N + 16 * _STRIPE, _TAIL)])


_deg_kernel = pl.kernel(_deg_body, **_DEG_KW)


_FACONV_KW = dict(
    out_type=jax.ShapeDtypeStruct((_NC, _N, _D), _f32),
    mesh=_MESH,
    compiler_params=pltpu.CompilerParams(needs_layout_passes=False),
    scratch_types=[
        pltpu.VMEM((_N,), _f32),
        pltpu.VMEM((_N,), _f32),
        pltpu.VMEM((_N,), _f32),
        pltpu.VMEM((_B,), _i32),
        pltpu.VMEM((_B,), _i32),
        pltpu.VMEM((_B,), _f32),
        pltpu.VMEM((_B, _D), _f32),
        pltpu.VMEM((16, _D), _f32),
        pltpu.SemaphoreType.DMA,
        pltpu.VMEM_SHARED((_N, _D), _f32),
    ],
)


def _faconv_body(h_hbm, src_hbm, dst_hbm, hl_hbm, hr_hbm, dis_hbm, part_out,
                   hl_v, hr_v, dis_v, srci_v, dsti_v, w_v, rows_v, zr_v,
                   gsem, out_sh):
    cid = lax.axis_index("c")
    sid = lax.axis_index("s")

    pltpu.sync_copy(hl_hbm, hl_v)
    pltpu.sync_copy(hr_hbm, hr_v)
    pltpu.sync_copy(dis_hbm, dis_v)

    for r in range(16):
        for f in range(_D // _LANES):
            zr_v[r, pl.ds(f * _LANES, _LANES)] = jnp.zeros((_LANES,), _f32)

    rbase = sid * _STRIPE

    def _zero(k, c):
        pltpu.sync_copy(zr_v, out_sh.at[pl.ds(rbase + k * 16, 16)])
        return c

    lax.fori_loop(0, _STRIPE // 16, _zero, 0)

    @pl.when(sid == _NS - 1)
    def _():
        pltpu.sync_copy(zr_v, out_sh.at[pl.ds(16 * _STRIPE, _TAIL)])

    plsc.subcore_barrier()

    base = (cid * _NS + sid) * _EW

    def _body(i, c):
        eb = base + i * _B
        pltpu.sync_copy(src_hbm.at[pl.ds(eb, _B)], srci_v)
        pltpu.sync_copy(dst_hbm.at[pl.ds(eb, _B)], dsti_v)
        pltpu.async_copy(h_hbm.at[srci_v], rows_v, gsem).wait()
        for g in range(_G):
            s16 = srci_v[pl.ds(g * _LANES, _LANES)]
            d16 = dsti_v[pl.ds(g * _LANES, _LANES)]
            hld = plsc.load_gather(hl_v, [d16])
            hrs = plsc.load_gather(hr_v, [s16])
            dss = plsc.load_gather(dis_v, [s16])
            dsd = plsc.load_gather(dis_v, [d16])
            y = hld + hrs
            e = jnp.exp(jnp.abs(y) * (-2.0))
            t = (1.0 - e) / (1.0 + e)
            t = jnp.where(y < 0.0, -t, t)
            w16 = t * dss * dsd
            lane = lax.iota(_i32, _LANES)
            for l in range(_LANES):
                wl_s = jnp.sum(jnp.where(lane == l, w16, 0.0))
                wl = jnp.full((_LANES,), wl_s)
                r = g * _LANES + l
                for f in range(_D // _LANES):
                    sl = pl.ds(f * _LANES, _LANES)
                    rows_v[r, sl] = rows_v[r, sl] * wl
        pltpu.sync_copy(rows_v, out_sh.at[dsti_v], add=True)
        return c

    lax.fori_loop(0, _NB, _body, 0)
    plsc.subcore_barrier()

    def _cpout(off, cnt):
        pltpu.sync_copy(out_sh.at[pl.ds(off, cnt)], rows_v.at[pl.ds(0, cnt)])
        pltpu.sync_copy(rows_v.at[pl.ds(0, cnt)],
                        part_out.at[cid, pl.ds(off, cnt)])

    for j in range(_STRIPE // _B):
        _cpout(rbase + j * _B, _B)
    _cpout(rbase + (_STRIPE // _B) * _B, _STRIPE % _B)

    @pl.when(sid == _NS - 1)
    def _():
        _cpout(16 * _STRIPE, _TAIL)


_faconv_kernel = pl.kernel(_faconv_body, **_FACONV_KW)


def _tc_a_body(x_ref, w1_ref, b1_ref, al_ref, ar_ref, degp_ref,
               h_ref, hl_ref, hr_ref, dis_ref):
    xw = lax.dot_general(x_ref[...], w1_ref[...], (((1,), (1,)), ((), ())),
                         preferred_element_type=_f32)
    h = jnp.maximum(xw + b1_ref[...][None, :], 0.0)
    h_ref[...] = h
    hl_ref[...] = jnp.sum(h * al_ref[...][None, :], axis=1)
    hr_ref[...] = jnp.sum(h * ar_ref[...][None, :], axis=1)
    dp = degp_ref[...]
    deg = dp[:_N] + dp[_N:]
    dis_ref[...] = jnp.where(deg > 0.0, 1.0 / jnp.sqrt(jnp.maximum(deg, 1.0)), 0.0)


def _tc_b_body(raw_ref, p_ref, al_ref, ar_ref, h_ref, hl_ref, hr_ref):
    h = _EPS * raw_ref[...] + p_ref[0] + p_ref[1]
    h_ref[...] = h
    hl_ref[...] = jnp.sum(h * al_ref[...][None, :], axis=1)
    hr_ref[...] = jnp.sum(h * ar_ref[...][None, :], axis=1)


def _tc_c_body(raw_ref, q_ref, w2_ref, b2_ref, o_ref):
    h = _EPS * raw_ref[...] + q_ref[0] + q_ref[1]
    z = lax.dot_general(h, w2_ref[...], (((1,), (1,)), ((), ())),
                        preferred_element_type=_f32) + b2_ref[...][None, :]
    m = jnp.max(z, axis=1, keepdims=True)
    s = jnp.log(jnp.sum(jnp.exp(z - m), axis=1, keepdims=True))
    o_ref[...] = z - m - s


_tc_a = pl.pallas_call(
    _tc_a_body,
    out_shape=[
        jax.ShapeDtypeStruct((_N, _D), _f32),
        jax.ShapeDtypeStruct((_N,), _f32),
        jax.ShapeDtypeStruct((_N,), _f32),
        jax.ShapeDtypeStruct((_N,), _f32),
    ],
)

_tc_b = pl.pallas_call(
    _tc_b_body,
    out_shape=[
        jax.ShapeDtypeStruct((_N, _D), _f32),
        jax.ShapeDtypeStruct((_N,), _f32),
        jax.ShapeDtypeStruct((_N,), _f32),
    ],
)

_tc_c = pl.pallas_call(
    _tc_c_body,
    out_shape=jax.ShapeDtypeStruct((_N, _D), _f32),
)


def kernel(x, edge_index, W1, b1, W2, b2, aL0, aR0, aL1, aR1):
    src = edge_index[0]
    dst = edge_index[1]
    degp = _deg_kernel(dst)
    h, hl0, hr0, dis = _tc_a(x, W1, b1, aL0, aR0, degp)
    p0 = _faconv_kernel(h, src, dst, hl0, hr0, dis)
    h1, hl1, hr1 = _tc_b(h, p0, aL1, aR1)
    p1 = _faconv_kernel(h1, src, dst, hl1, hr1, dis)
    return _tc_c(h, p1, W2, b2)


# speedup vs baseline: 18.5857x; 1.7019x over previous
import functools

import numpy as np

import jax
import jax.numpy as jnp
from jax import lax
from jax.experimental import pallas as pl
from jax.experimental.pallas import tpu as pltpu
from jax.experimental.pallas import tpu_sc as plsc

_N = 10000
_E = 320000
_D = 128
_EPS = 0.3

_NC = 2
_NS = 16
_LANES = 16
_NW = _NC * _NS
_EW = _E // _NW
_B = 80
_NB = _EW // _B
_G = _B // _LANES

_STRIPE = 624
_TAIL = _N - 16 * _STRIPE
_ROWS_PER_TILE = _N // _NS

_f32 = jnp.float32
_i32 = jnp.int32

_MESH = plsc.VectorSubcoreMesh(core_axis_name="c", subcore_axis_name="s",
                               num_cores=_NC, num_subcores=_NS)


_DEG_KW = dict(
    out_type=jax.ShapeDtypeStruct((_NC * _N,), _f32),
    mesh=_MESH,
    compiler_params=pltpu.CompilerParams(needs_layout_passes=False),
    scratch_types=[
        pltpu.VMEM((_B,), _i32),
        pltpu.VMEM((_B,), _f32),
        pltpu.VMEM((_STRIPE,), _f32),
        pltpu.VMEM_SHARED((_N,), _f32),
    ],
)


def _deg_body(dst_hbm, deg_out, dsti_v, ones_v, zb_v, deg_sh):
    cid = lax.axis_index("c")
    sid = lax.axis_index("s")
    for j in range(_B // _LANES):
        ones_v[pl.ds(j * _LANES, _LANES)] = jnp.full((_LANES,), 1.0, _f32)

    def _zfill(j, c):
        zb_v[pl.ds(j * _LANES, _LANES)] = jnp.zeros((_LANES,), _f32)
        return c

    lax.fori_loop(0, _STRIPE // _LANES, _zfill, 0)
    pltpu.sync_copy(zb_v, deg_sh.at[pl.ds(sid * _STRIPE, _STRIPE)])

    @pl.when(sid == _NS - 1)
    def _():
        pltpu.sync_copy(zb_v.at[pl.ds(0, _TAIL)],
                        deg_sh.at[pl.ds(16 * _STRIPE, _TAIL)])

    plsc.subcore_barrier()

    base = (cid * _NS + sid) * _EW

    def _body(i, c):
        pltpu.sync_copy(dst_hbm.at[pl.ds(base + i * _B, _B)], dsti_v)
        pltpu.sync_copy(ones_v, deg_sh.at[dsti_v], add=True)
        return c

    lax.fori_loop(0, _NB, _body, 0)
    plsc.subcore_barrier()

    pltpu.sync_copy(deg_sh.at[pl.ds(sid * _STRIPE, _STRIPE)], zb_v)
    pltpu.sync_copy(zb_v, deg_out.at[pl.ds(cid * _N + sid * _STRIPE, _STRIPE)])

    @pl.when(sid == _NS - 1)
    def _():
        pltpu.sync_copy(deg_sh.at[pl.ds(16 * _STRIPE, _TAIL)],
                        ones_v.at[pl.ds(0, _TAIL)])
        pltpu.sync_copy(ones_v.at[pl.ds(0, _TAIL)],
                        deg_out.at[pl.ds(cid * _N + 16 * _STRIPE, _TAIL)])


_deg_kernel = pl.kernel(_deg_body, **_DEG_KW)


_B2 = 128
_NB2 = _EW // _B2
_ETAIL = _EW - _NB2 * _B2
_G2 = _B2 // _LANES

_FACONV_KW = dict(
    out_type=jax.ShapeDtypeStruct((_NC, _N, _D), _f32),
    mesh=_MESH,
    compiler_params=pltpu.CompilerParams(needs_layout_passes=False),
    scratch_types=[
        pltpu.VMEM((_B2,), _i32),
        pltpu.VMEM((_B2,), _i32),
        pltpu.VMEM((_B2,), _i32),
        pltpu.VMEM((_B2,), _i32),
        pltpu.VMEM((_B2, _D), _f32),
        pltpu.VMEM((_B2, _D), _f32),
        pltpu.VMEM((_B2,), _f32),
        pltpu.VMEM((_B2,), _f32),
        pltpu.VMEM((_B2,), _f32),
        pltpu.VMEM((_B2,), _f32),
        pltpu.VMEM((_B2,), _f32),
        pltpu.VMEM((_B2,), _f32),
        pltpu.VMEM((_B2,), _f32),
        pltpu.VMEM((_B2,), _f32),
        pltpu.VMEM((_ETAIL,), _i32),
        pltpu.VMEM((_ETAIL,), _i32),
        pltpu.VMEM((_ETAIL,), _f32),
        pltpu.VMEM((_ETAIL,), _f32),
        pltpu.VMEM((_ETAIL,), _f32),
        pltpu.VMEM((_ETAIL,), _f32),
        pltpu.VMEM((16, _D), _f32),
        pltpu.SemaphoreType.DMA,
        pltpu.SemaphoreType.DMA,
        pltpu.SemaphoreType.DMA,
        pltpu.SemaphoreType.DMA,
        pltpu.SemaphoreType.DMA,
        pltpu.SemaphoreType.DMA,
        pltpu.VMEM_SHARED((_N, _D), _f32),
    ],
)


def _faconv_body(h_hbm, src_hbm, dst_hbm, hl_hbm, hr_hbm, dis_hbm, part_out,
                 srci0, srci1, dsti0, dsti1, rows0, rows1,
                 hld0, hld1, hrs0, hrs1, dss0, dss1, dsd0, dsd1,
                 srct_v, dstt_v, hldt, hrst, dsst, dsdt,
                 zr_v, ss0, ss1, ds0, ds1, gs0, gs1, out_sh):
    cid = lax.axis_index("c")
    sid = lax.axis_index("s")

    for r in range(16):
        for f in range(_D // _LANES):
            zr_v[r, pl.ds(f * _LANES, _LANES)] = jnp.zeros((_LANES,), _f32)

    rbase = sid * _STRIPE

    def _zero(k, c):
        pltpu.sync_copy(zr_v, out_sh.at[pl.ds(rbase + k * 16, 16)])
        return c

    lax.fori_loop(0, _STRIPE // 16, _zero, 0)

    @pl.when(sid == _NS - 1)
    def _():
        pltpu.sync_copy(zr_v, out_sh.at[pl.ds(16 * _STRIPE, _TAIL)])

    base = (cid * _NS + sid) * _EW
    slots = ((srci0, dsti0, rows0, hld0, hrs0, dss0, dsd0, ss0, ds0, gs0),
             (srci1, dsti1, rows1, hld1, hrs1, dss1, dsd1, ss1, ds1, gs1))

    def _issue_idx(i, s):
        eb = base + i * _B2
        sr, dr = slots[s][0], slots[s][1]
        ssem, dsem = slots[s][7], slots[s][8]
        pltpu.async_copy(src_hbm.at[pl.ds(eb, _B2)], sr, ssem)
        pltpu.async_copy(dst_hbm.at[pl.ds(eb, _B2)], dr, dsem)

    def _wait_idx(s):
        sr, dr = slots[s][0], slots[s][1]
        ssem, dsem = slots[s][7], slots[s][8]
        pltpu.make_async_copy(src_hbm.at[pl.ds(base, _B2)], sr, ssem).wait()
        pltpu.make_async_copy(dst_hbm.at[pl.ds(base, _B2)], dr, dsem).wait()

    def _issue_gather(s):
        sr, dr, rw, hl_, hr_, dss_, dsd_, _, _, gsem = slots[s]
        pltpu.async_copy(h_hbm.at[sr], rw, gsem)
        pltpu.async_copy(hl_hbm.at[dr], hl_, gsem)
        pltpu.async_copy(hr_hbm.at[sr], hr_, gsem)
        pltpu.async_copy(dis_hbm.at[sr], dss_, gsem)
        pltpu.async_copy(dis_hbm.at[dr], dsd_, gsem)

    def _wait_gather(s):
        sr, dr, rw, hl_, hr_, dss_, dsd_, _, _, gsem = slots[s]
        pltpu.make_async_copy(h_hbm.at[sr], rw, gsem).wait()
        pltpu.make_async_copy(hl_hbm.at[dr], hl_, gsem).wait()
        pltpu.make_async_copy(hr_hbm.at[sr], hr_, gsem).wait()
        pltpu.make_async_copy(dis_hbm.at[sr], dss_, gsem).wait()
        pltpu.make_async_copy(dis_hbm.at[dr], dsd_, gsem).wait()

    def _scale_group(hl_, hr_, dss_, dsd_, rw, g):
        sl16 = pl.ds(g * _LANES, _LANES)
        y = hl_[sl16] + hr_[sl16]
        e = jnp.exp(jnp.abs(y) * (-2.0))
        t = (1.0 - e) / (1.0 + e)
        t = jnp.where(y < 0.0, -t, t)
        w16 = t * dss_[sl16] * dsd_[sl16]
        lane = lax.iota(_i32, _LANES)
        for l in range(_LANES):
            wl_s = jnp.sum(jnp.where(lane == l, w16, 0.0))
            wl = jnp.full((_LANES,), wl_s)
            r = g * _LANES + l
            for f in range(_D // _LANES):
                sl = pl.ds(f * _LANES, _LANES)
                rw[r, sl] = rw[r, sl] * wl

    def _compute_scatter(s):
        dr, rw = slots[s][1], slots[s][2]
        hl_, hr_, dss_, dsd_ = slots[s][3:7]

        def _grp(g, c):
            _scale_group(hl_, hr_, dss_, dsd_, rw, g)
            return c

        lax.fori_loop(0, _G2, _grp, 0)
        pltpu.sync_copy(rw, out_sh.at[dr], add=True)

    _issue_idx(0, 0)
    _wait_idx(0)
    _issue_gather(0)
    _issue_idx(1, 1)
    plsc.subcore_barrier()

    def _pipe(i, s, prefetch):
        _wait_idx(1 - s)
        _issue_gather(1 - s)
        _wait_gather(s)
        _compute_scatter(s)
        if prefetch:
            _issue_idx(i + 2, s)

    def _pair(k, c):
        i = 2 * k
        _pipe(i, 0, True)
        _pipe(i + 1, 1, True)
        return c

    lax.fori_loop(0, _NB2 // 2 - 1, _pair, 0)
    _wait_idx(1)
    _issue_gather(1)
    _wait_gather(0)
    _compute_scatter(0)
    _wait_gather(1)
    _compute_scatter(1)

    tb = base + _NB2 * _B2
    pltpu.sync_copy(src_hbm.at[pl.ds(tb, _ETAIL)], srct_v)
    pltpu.sync_copy(dst_hbm.at[pl.ds(tb, _ETAIL)], dstt_v)
    pltpu.async_copy(h_hbm.at[srct_v], zr_v, gs0)
    pltpu.async_copy(hl_hbm.at[dstt_v], hldt, gs0)
    pltpu.async_copy(hr_hbm.at[srct_v], hrst, gs0)
    pltpu.async_copy(dis_hbm.at[srct_v], dsst, gs0)
    pltpu.async_copy(dis_hbm.at[dstt_v], dsdt, gs0)
    pltpu.make_async_copy(h_hbm.at[srct_v], zr_v, gs0).wait()
    pltpu.make_async_copy(hl_hbm.at[dstt_v], hldt, gs0).wait()
    pltpu.make_async_copy(hr_hbm.at[srct_v], hrst, gs0).wait()
    pltpu.make_async_copy(dis_hbm.at[srct_v], dsst, gs0).wait()
    pltpu.make_async_copy(dis_hbm.at[dstt_v], dsdt, gs0).wait()
    _scale_group(hldt, hrst, dsst, dsdt, zr_v, 0)
    pltpu.sync_copy(zr_v, out_sh.at[dstt_v], add=True)

    plsc.subcore_barrier()

    def _cpout(off, cnt):
        pltpu.sync_copy(out_sh.at[pl.ds(off, cnt)], rows0.at[pl.ds(0, cnt)])
        pltpu.sync_copy(rows0.at[pl.ds(0, cnt)],
                        part_out.at[cid, pl.ds(off, cnt)])

    for j in range(_STRIPE // _B2):
        _cpout(rbase + j * _B2, _B2)
    _cpout(rbase + (_STRIPE // _B2) * _B2, _STRIPE % _B2)

    @pl.when(sid == _NS - 1)
    def _():
        _cpout(16 * _STRIPE, _TAIL)


_faconv_kernel = pl.kernel(_faconv_body, **_FACONV_KW)


def _tc_a_body(x_ref, w1_ref, b1_ref, al_ref, ar_ref, degp_ref,
               h_ref, hl_ref, hr_ref, dis_ref):
    xw = lax.dot_general(x_ref[...], w1_ref[...], (((1,), (1,)), ((), ())),
                         preferred_element_type=_f32)
    h = jnp.maximum(xw + b1_ref[...][None, :], 0.0)
    h_ref[...] = h
    hl_ref[...] = jnp.sum(h * al_ref[...][None, :], axis=1)
    hr_ref[...] = jnp.sum(h * ar_ref[...][None, :], axis=1)
    dp = degp_ref[...]
    deg = dp[:_N] + dp[_N:]
    dis_ref[...] = jnp.where(deg > 0.0, 1.0 / jnp.sqrt(jnp.maximum(deg, 1.0)), 0.0)


def _tc_b_body(raw_ref, p_ref, al_ref, ar_ref, h_ref, hl_ref, hr_ref):
    h = _EPS * raw_ref[...] + p_ref[0] + p_ref[1]
    h_ref[...] = h
    hl_ref[...] = jnp.sum(h * al_ref[...][None, :], axis=1)
    hr_ref[...] = jnp.sum(h * ar_ref[...][None, :], axis=1)


def _tc_c_body(raw_ref, q_ref, w2_ref, b2_ref, o_ref):
    h = _EPS * raw_ref[...] + q_ref[0] + q_ref[1]
    z = lax.dot_general(h, w2_ref[...], (((1,), (1,)), ((), ())),
                        preferred_element_type=_f32) + b2_ref[...][None, :]
    m = jnp.max(z, axis=1, keepdims=True)
    s = jnp.log(jnp.sum(jnp.exp(z - m), axis=1, keepdims=True))
    o_ref[...] = z - m - s


_tc_a = pl.pallas_call(
    _tc_a_body,
    out_shape=[
        jax.ShapeDtypeStruct((_N, _D), _f32),
        jax.ShapeDtypeStruct((_N,), _f32),
        jax.ShapeDtypeStruct((_N,), _f32),
        jax.ShapeDtypeStruct((_N,), _f32),
    ],
)

_tc_b = pl.pallas_call(
    _tc_b_body,
    out_shape=[
        jax.ShapeDtypeStruct((_N, _D), _f32),
        jax.ShapeDtypeStruct((_N,), _f32),
        jax.ShapeDtypeStruct((_N,), _f32),
    ],
)

_tc_c = pl.pallas_call(
    _tc_c_body,
    out_shape=jax.ShapeDtypeStruct((_N, _D), _f32),
)


def kernel(x, edge_index, W1, b1, W2, b2, aL0, aR0, aL1, aR1):
    src = edge_index[0]
    dst = edge_index[1]
    degp = _deg_kernel(dst)
    h, hl0, hr0, dis = _tc_a(x, W1, b1, aL0, aR0, degp)
    p0 = _faconv_kernel(h, src, dst, hl0, hr0, dis)
    h1, hl1, hr1 = _tc_b(h, p0, aL1, aR1)
    p1 = _faconv_kernel(h1, src, dst, hl1, hr1, dis)
    return _tc_c(h, p1, W2, b2)


# speedup vs baseline: 18.5975x; 1.0006x over previous
import functools

import numpy as np

import jax
import jax.numpy as jnp
from jax import lax
from jax.experimental import pallas as pl
from jax.experimental.pallas import tpu as pltpu
from jax.experimental.pallas import tpu_sc as plsc

_N = 10000
_E = 320000
_D = 128
_EPS = 0.3

_NC = 2
_NS = 16
_LANES = 16
_NW = _NC * _NS
_EW = _E // _NW
_B = 80
_NB = _EW // _B
_G = _B // _LANES

_STRIPE = 624
_TAIL = _N - 16 * _STRIPE
_ROWS_PER_TILE = _N // _NS

_f32 = jnp.float32
_i32 = jnp.int32

_MESH = plsc.VectorSubcoreMesh(core_axis_name="c", subcore_axis_name="s",
                               num_cores=_NC, num_subcores=_NS)


_DEG_KW = dict(
    out_type=jax.ShapeDtypeStruct((_NC * _N,), _f32),
    mesh=_MESH,
    compiler_params=pltpu.CompilerParams(needs_layout_passes=False),
    scratch_types=[
        pltpu.VMEM((_B,), _i32),
        pltpu.VMEM((_B,), _f32),
        pltpu.VMEM((_STRIPE,), _f32),
        pltpu.VMEM_SHARED((_N,), _f32),
    ],
)


def _deg_body(dst_hbm, deg_out, dsti_v, ones_v, zb_v, deg_sh):
    cid = lax.axis_index("c")
    sid = lax.axis_index("s")
    for j in range(_B // _LANES):
        ones_v[pl.ds(j * _LANES, _LANES)] = jnp.full((_LANES,), 1.0, _f32)

    def _zfill(j, c):
        zb_v[pl.ds(j * _LANES, _LANES)] = jnp.zeros((_LANES,), _f32)
        return c

    lax.fori_loop(0, _STRIPE // _LANES, _zfill, 0)
    pltpu.sync_copy(zb_v, deg_sh.at[pl.ds(sid * _STRIPE, _STRIPE)])

    @pl.when(sid == _NS - 1)
    def _():
        pltpu.sync_copy(zb_v.at[pl.ds(0, _TAIL)],
                        deg_sh.at[pl.ds(16 * _STRIPE, _TAIL)])

    plsc.subcore_barrier()

    base = (cid * _NS + sid) * _EW

    def _body(i, c):
        pltpu.sync_copy(dst_hbm.at[pl.ds(base + i * _B, _B)], dsti_v)
        pltpu.sync_copy(ones_v, deg_sh.at[dsti_v], add=True)
        return c

    lax.fori_loop(0, _NB, _body, 0)
    plsc.subcore_barrier()

    pltpu.sync_copy(deg_sh.at[pl.ds(sid * _STRIPE, _STRIPE)], zb_v)
    pltpu.sync_copy(zb_v, deg_out.at[pl.ds(cid * _N + sid * _STRIPE, _STRIPE)])

    @pl.when(sid == _NS - 1)
    def _():
        pltpu.sync_copy(deg_sh.at[pl.ds(16 * _STRIPE, _TAIL)],
                        ones_v.at[pl.ds(0, _TAIL)])
        pltpu.sync_copy(ones_v.at[pl.ds(0, _TAIL)],
                        deg_out.at[pl.ds(cid * _N + 16 * _STRIPE, _TAIL)])


_deg_kernel = pl.kernel(_deg_body, **_DEG_KW)


_B2 = 96
_NB2 = _EW // _B2
_ETAIL = _EW - _NB2 * _B2
_G2 = _B2 // _LANES
_NSLOT = 3

_SLOT_SCRATCH = [
    pltpu.VMEM((_B2,), _i32),
    pltpu.VMEM((_B2,), _i32),
    pltpu.VMEM((_B2, _D), _f32),
    pltpu.VMEM((_B2,), _f32),
    pltpu.VMEM((_B2,), _f32),
    pltpu.VMEM((_B2,), _f32),
    pltpu.VMEM((_B2,), _f32),
    pltpu.VMEM((_B2,), _i32),
    pltpu.SemaphoreType.DMA,
    pltpu.SemaphoreType.DMA,
    pltpu.SemaphoreType.DMA,
    pltpu.SemaphoreType.DMA,
]

_FACONV_KW = dict(
    out_type=jax.ShapeDtypeStruct((_NC, _N, _D), _f32),
    mesh=_MESH,
    compiler_params=pltpu.CompilerParams(needs_layout_passes=False),
    scratch_types=(
        _SLOT_SCRATCH * _NSLOT
        + [
            pltpu.VMEM((_ETAIL,), _i32),
            pltpu.VMEM((_ETAIL,), _i32),
            pltpu.VMEM((_ETAIL,), _f32),
            pltpu.VMEM((_ETAIL,), _f32),
            pltpu.VMEM((_ETAIL,), _f32),
            pltpu.VMEM((_ETAIL,), _f32),
            pltpu.VMEM((48, _D), _f32),
            pltpu.VMEM_SHARED((_N, _D), _f32),
        ]
    ),
)


def _faconv_body(h_hbm, src_hbm, dst_hbm, hl_hbm, hr_hbm, dis_hbm, part_out,
                 *scratch):
    nslot_args = len(_SLOT_SCRATCH)
    slots = tuple(scratch[k * nslot_args:(k + 1) * nslot_args]
                  for k in range(_NSLOT))
    (srct_v, dstt_v, hldt, hrst, dsst, dsdt, zr_v, out_sh) = \
        scratch[_NSLOT * nslot_args:]
    rows0 = slots[0][2]
    cid = lax.axis_index("c")
    sid = lax.axis_index("s")

    def _zfill(r, c):
        for f in range(_D // _LANES):
            zr_v[r, pl.ds(f * _LANES, _LANES)] = jnp.zeros((_LANES,), _f32)
        return c

    lax.fori_loop(0, 48, _zfill, 0)

    rbase = sid * _STRIPE

    def _zero(k, c):
        pltpu.sync_copy(zr_v, out_sh.at[pl.ds(rbase + k * 48, 48)])
        return c

    lax.fori_loop(0, _STRIPE // 48, _zero, 0)

    @pl.when(sid == _NS - 1)
    def _():
        pltpu.sync_copy(zr_v.at[pl.ds(0, _TAIL)],
                        out_sh.at[pl.ds(16 * _STRIPE, _TAIL)])

    base = (cid * _NS + sid) * _EW

    def _issue_idx(i, s):
        eb = base + i * _B2
        sr, dr, ssem, dsem = slots[s][0], slots[s][1], slots[s][8], slots[s][9]
        pltpu.async_copy(src_hbm.at[pl.ds(eb, _B2)], sr, ssem)
        pltpu.async_copy(dst_hbm.at[pl.ds(eb, _B2)], dr, dsem)

    def _wait_idx(s):
        sr, dr, ssem, dsem = slots[s][0], slots[s][1], slots[s][8], slots[s][9]
        pltpu.make_async_copy(src_hbm.at[pl.ds(base, _B2)], sr, ssem).wait()
        pltpu.make_async_copy(dst_hbm.at[pl.ds(base, _B2)], dr, dsem).wait()

    def _issue_gather(s):
        sr, dr, rw, hl_, hr_, dss_, dsd_ = slots[s][:7]
        gsem = slots[s][10]
        pltpu.async_copy(h_hbm.at[sr], rw, gsem)
        pltpu.async_copy(hl_hbm.at[dr], hl_, gsem)
        pltpu.async_copy(hr_hbm.at[sr], hr_, gsem)
        pltpu.async_copy(dis_hbm.at[sr], dss_, gsem)
        pltpu.async_copy(dis_hbm.at[dr], dsd_, gsem)

    def _wait_gather(s):
        sr, dr, rw, hl_, hr_, dss_, dsd_ = slots[s][:7]
        gsem = slots[s][10]
        pltpu.make_async_copy(h_hbm.at[sr], rw, gsem).wait()
        pltpu.make_async_copy(hl_hbm.at[dr], hl_, gsem).wait()
        pltpu.make_async_copy(hr_hbm.at[sr], hr_, gsem).wait()
        pltpu.make_async_copy(dis_hbm.at[sr], dss_, gsem).wait()
        pltpu.make_async_copy(dis_hbm.at[dr], dsd_, gsem).wait()

    def _wait_scatter(s):
        rw, drs, scsem = slots[s][2], slots[s][7], slots[s][11]
        pltpu.make_async_copy(rw, out_sh.at[drs], scsem).wait()

    def _scale_group(hl_, hr_, dss_, dsd_, rw, g):
        sl16 = pl.ds(g * _LANES, _LANES)
        y = hl_[sl16] + hr_[sl16]
        e = jnp.exp(jnp.abs(y) * (-2.0))
        t = (1.0 - e) / (1.0 + e)
        t = jnp.where(y < 0.0, -t, t)
        w16 = t * dss_[sl16] * dsd_[sl16]
        lane = lax.iota(_i32, _LANES)
        for l in range(_LANES):
            wl_s = jnp.sum(jnp.where(lane == l, w16, 0.0))
            wl = jnp.full((_LANES,), wl_s)
            r = g * _LANES + l
            for f in range(_D // _LANES):
                sl = pl.ds(f * _LANES, _LANES)
                rw[r, sl] = rw[r, sl] * wl

    def _compute_scatter(s):
        dr, rw = slots[s][1], slots[s][2]
        hl_, hr_, dss_, dsd_ = slots[s][3:7]
        drs, scsem = slots[s][7], slots[s][11]

        def _grp(g, c):
            _scale_group(hl_, hr_, dss_, dsd_, rw, g)
            return c

        lax.fori_loop(0, _G2, _grp, 0)
        for g in range(_G2):
            sl = pl.ds(g * _LANES, _LANES)
            drs[sl] = dr[sl]
        pltpu.async_copy(rw, out_sh.at[drs], scsem, add=True)

    _issue_idx(0, 0)
    _wait_idx(0)
    _issue_gather(0)
    _issue_idx(1, 1)
    plsc.subcore_barrier()

    def _pipe(i, s, wait_sc=True, next_gather=True, prefetch=True):
        ns = (s + 1) % _NSLOT
        if next_gather:
            _wait_idx(ns)
            if wait_sc:
                _wait_scatter(ns)
            _issue_gather(ns)
        _wait_gather(s)
        _compute_scatter(s)
        if prefetch:
            _issue_idx(i + 2, (s + 2) % _NSLOT)

    _pipe(0, 0, wait_sc=False)
    _pipe(1, 1, wait_sc=False)

    def _triple(k, c):
        i = 3 * k + 2
        _pipe(i, 2)
        _pipe(i + 1, 0)
        _pipe(i + 2, 1)
        return c

    lax.fori_loop(0, (_NB2 - 5) // 3, _triple, 0)
    _pipe(_NB2 - 3, (_NB2 - 3) % _NSLOT)
    _pipe(_NB2 - 2, (_NB2 - 2) % _NSLOT, prefetch=False)
    _pipe(_NB2 - 1, (_NB2 - 1) % _NSLOT, next_gather=False, prefetch=False)
    _wait_scatter((_NB2 - 3) % _NSLOT)
    _wait_scatter((_NB2 - 2) % _NSLOT)
    _wait_scatter((_NB2 - 1) % _NSLOT)

    tb = base + _NB2 * _B2
    gs0 = slots[0][10]
    zt = zr_v.at[pl.ds(0, _ETAIL)]
    pltpu.sync_copy(src_hbm.at[pl.ds(tb, _ETAIL)], srct_v)
    pltpu.sync_copy(dst_hbm.at[pl.ds(tb, _ETAIL)], dstt_v)
    pltpu.async_copy(h_hbm.at[srct_v], zt, gs0)
    pltpu.async_copy(hl_hbm.at[dstt_v], hldt, gs0)
    pltpu.async_copy(hr_hbm.at[srct_v], hrst, gs0)
    pltpu.async_copy(dis_hbm.at[srct_v], dsst, gs0)
    pltpu.async_copy(dis_hbm.at[dstt_v], dsdt, gs0)
    pltpu.make_async_copy(h_hbm.at[srct_v], zt, gs0).wait()
    pltpu.make_async_copy(hl_hbm.at[dstt_v], hldt, gs0).wait()
    pltpu.make_async_copy(hr_hbm.at[srct_v], hrst, gs0).wait()
    pltpu.make_async_copy(dis_hbm.at[srct_v], dsst, gs0).wait()
    pltpu.make_async_copy(dis_hbm.at[dstt_v], dsdt, gs0).wait()
    _scale_group(hldt, hrst, dsst, dsdt, zr_v, 0)
    pltpu.sync_copy(zt, out_sh.at[dstt_v], add=True)

    plsc.subcore_barrier()

    def _cpout(off, cnt):
        pltpu.sync_copy(out_sh.at[pl.ds(off, cnt)], rows0.at[pl.ds(0, cnt)])
        pltpu.sync_copy(rows0.at[pl.ds(0, cnt)],
                        part_out.at[cid, pl.ds(off, cnt)])

    for j in range(_STRIPE // _B2):
        _cpout(rbase + j * _B2, _B2)
    _cpout(rbase + (_STRIPE // _B2) * _B2, _STRIPE % _B2)

    @pl.when(sid == _NS - 1)
    def _():
        _cpout(16 * _STRIPE, _TAIL)


_faconv_kernel = pl.kernel(_faconv_body, **_FACONV_KW)


def _tc_a_body(x_ref, w1_ref, b1_ref, al_ref, ar_ref, degp_ref,
               h_ref, hl_ref, hr_ref, dis_ref):
    xw = lax.dot_general(x_ref[...], w1_ref[...], (((1,), (1,)), ((), ())),
                         preferred_element_type=_f32)
    h = jnp.maximum(xw + b1_ref[...][None, :], 0.0)
    h_ref[...] = h
    hl_ref[...] = jnp.sum(h * al_ref[...][None, :], axis=1)
    hr_ref[...] = jnp.sum(h * ar_ref[...][None, :], axis=1)
    dp = degp_ref[...]
    deg = dp[:_N] + dp[_N:]
    dis_ref[...] = jnp.where(deg > 0.0, 1.0 / jnp.sqrt(jnp.maximum(deg, 1.0)), 0.0)


def _tc_b_body(raw_ref, p_ref, al_ref, ar_ref, h_ref, hl_ref, hr_ref):
    h = _EPS * raw_ref[...] + p_ref[0] + p_ref[1]
    h_ref[...] = h
    hl_ref[...] = jnp.sum(h * al_ref[...][None, :], axis=1)
    hr_ref[...] = jnp.sum(h * ar_ref[...][None, :], axis=1)


def _tc_c_body(raw_ref, q_ref, w2_ref, b2_ref, o_ref):
    h = _EPS * raw_ref[...] + q_ref[0] + q_ref[1]
    z = lax.dot_general(h, w2_ref[...], (((1,), (1,)), ((), ())),
                        preferred_element_type=_f32) + b2_ref[...][None, :]
    m = jnp.max(z, axis=1, keepdims=True)
    s = jnp.log(jnp.sum(jnp.exp(z - m), axis=1, keepdims=True))
    o_ref[...] = z - m - s


_tc_a = pl.pallas_call(
    _tc_a_body,
    out_shape=[
        jax.ShapeDtypeStruct((_N, _D), _f32),
        jax.ShapeDtypeStruct((_N,), _f32),
        jax.ShapeDtypeStruct((_N,), _f32),
        jax.ShapeDtypeStruct((_N,), _f32),
    ],
)

_tc_b = pl.pallas_call(
    _tc_b_body,
    out_shape=[
        jax.ShapeDtypeStruct((_N, _D), _f32),
        jax.ShapeDtypeStruct((_N,), _f32),
        jax.ShapeDtypeStruct((_N,), _f32),
    ],
)

_tc_c = pl.pallas_call(
    _tc_c_body,
    out_shape=jax.ShapeDtypeStruct((_N, _D), _f32),
)


def kernel(x, edge_index, W1, b1, W2, b2, aL0, aR0, aL1, aR1):
    src = edge_index[0]
    dst = edge_index[1]
    degp = _deg_kernel(dst)
    h, hl0, hr0, dis = _tc_a(x, W1, b1, aL0, aR0, degp)
    p0 = _faconv_kernel(h, src, dst, hl0, hr0, dis)
    h1, hl1, hr1 = _tc_b(h, p0, aL1, aR1)
    p1 = _faconv_kernel(h1, src, dst, hl1, hr1, dis)
    return _tc_c(h, p1, W2, b2)
